# baseline (speedup 1.0000x reference)
import functools

import jax
import jax.numpy as jnp
import numpy as np
from jax import lax
from jax.experimental import pallas as pl
from jax.experimental.pallas import tpu as pltpu
from jax.experimental.pallas import tpu_sc as plsc

N = 10000
E = 320000
F = 128

AVG_LOG = float(np.log(33.0))

NC = 2
NS = 16
NW = NC * NS
L = 16

NPAD = 10240
RPW = NPAD // NW
CR = RPW // 2
ACCR = CR + 8
DEGR = 16
SBLK = 2560
NSB = E // SBLK
STAGE = 2048
GB = 64
STCAP = STAGE + 2 * GB
ECAP = 158 * STAGE
SENT = CR << 14
SENTA = RPW << 14


def _accum_chunk(wid, k, nbat, b_hbm, bucket, sum_o, sq_o, mx_o, mn_o, deg_o,
                 sums, sqs, mxs, mns, degacc,
                 ebufA, sidxA, dbufA, rowA, dsemA,
                 ebufB, sidxB, dbufB, rowB, dsemB):
    zero = jnp.zeros((L,), jnp.float32)
    neg = jnp.full((L,), -3.0e38, jnp.float32)
    pos = jnp.full((L,), 3.0e38, jnp.float32)
    lane = lax.iota(jnp.int32, L)
    bbase = (wid * 3 + k) * ECAP

    @pl.loop(0, DEGR)
    def _initd(r):
        degacc[r, :] = zero

    @pl.loop(0, ACCR)
    def _init(r):
        rb = r * F
        for j in range(F // L):
            sl = pl.ds(rb + j * L, L)
            sums[sl] = zero
            sqs[sl] = zero
            mxs[sl] = neg
            mns[sl] = pos

    def fetch(b, eb, sx, dbf, row, sem):
        off = pl.multiple_of(bbase + b * GB, GB)
        pltpu.sync_copy(bucket.at[pl.ds(off, GB)], eb)
        for i in range(GB // L):
            v = eb[pl.ds(i * L, L)]
            sx[pl.ds(i * L, L)] = jnp.minimum(v & 16383, NPAD - 1)
            dbf[pl.ds(i * L, L)] = lax.shift_right_logical(v, 14)
        pltpu.async_copy(b_hbm.at[sx], row, sem)

    def wait_row(row, sem):
        pltpu.make_async_copy(b_hbm.at[pl.ds(0, GB)], row, sem).wait()

    def process(dbf, row):
        @pl.loop(0, GB // L)
        def _grp(gi):
            dv = dbf[pl.ds(gi * L, L)]
            for l in range(L):
                dle = dv[l]
                e = gi * L + l
                db = dle * F
                dg = lax.shift_right_logical(dle, 4)
                onehot = jnp.where(lane == (dle & 15), 1.0, 0.0)
                degacc[dg, :] = degacc[dg, :] + onehot
                for j in range(F // L):
                    sl = pl.ds(db + j * L, L)
                    r = row[e, pl.ds(j * L, L)]
                    sums[sl] = sums[sl] + r
                    sqs[sl] = sqs[sl] + r * r
                    mxs[sl] = jnp.maximum(mxs[sl], r)
                    mns[sl] = jnp.minimum(mns[sl], r)

    fetch(0, ebufA, sidxA, dbufA, rowA, dsemA)

    def pair(p, x):
        fetch(2 * p + 1, ebufB, sidxB, dbufB, rowB, dsemB)
        wait_row(rowA, dsemA)
        process(dbufA, rowA)
        fetch(2 * p + 2, ebufA, sidxA, dbufA, rowA, dsemA)
        wait_row(rowB, dsemB)
        process(dbufB, rowB)
        return x

    lax.fori_loop(0, lax.div(nbat, 2), pair, 0)
    wait_row(rowA, dsemA)

    base = pl.multiple_of((wid * RPW + k * CR) * F, CR * F)
    pltpu.sync_copy(sums.at[pl.ds(0, CR * F)], sum_o.at[pl.ds(base, CR * F)])
    pltpu.sync_copy(sqs.at[pl.ds(0, CR * F)], sq_o.at[pl.ds(base, CR * F)])
    pltpu.sync_copy(mxs.at[pl.ds(0, CR * F)], mx_o.at[pl.ds(base, CR * F)])
    pltpu.sync_copy(mns.at[pl.ds(0, CR * F)], mn_o.at[pl.ds(base, CR * F)])
    dbase = pl.multiple_of((wid * 2 + k) * DEGR, DEGR)
    pltpu.sync_copy(degacc.at[pl.ds(0, DEGR)], deg_o.at[pl.ds(dbase, DEGR)])


def _scan_compact(wid, src_hbm, dst_hbm, bucket, srcA, dstA, srcB, dstB,
                  sas, sad, sbs, sbd, stage0, stage1, tmp, qblkbuf):
    lo = wid * RPW
    lane = lax.iota(jnp.int32, L)
    zero = jnp.zeros((L,), jnp.int32)
    stage_v = jnp.full((L,), STAGE, jnp.int32)
    sentv = jnp.full((L,), SENT, jnp.int32)
    sentav = jnp.full((L,), SENTA, jnp.int32)
    tmp[pl.ds(0, L)] = zero
    qbase = (wid * 3 + 2) * ECAP

    def compact(mi, values, stg, qp, qv, g, gv, base):
        acc = mi
        for sh in (1, 2, 4, 8):
            tmp[pl.ds(L, L)] = acc
            acc = acc + tmp[pl.ds(L - sh, L)]
        target = lane + 1
        posv = zero
        for stp in (8, 4, 2, 1):
            cand = posv + stp
            pv = acc[(cand - 1) & 15]
            posv = jnp.where((pv < target) & (cand <= L), cand, posv)
        stg[pl.ds(qp, L)] = values[posv & 15]
        c15 = acc[15]
        qp2 = qp + c15
        qv2 = qv + c15
        fi = lax.div(qp2, STAGE)
        fiv = lax.div(qv2, stage_v)

        def flush(x):
            boff = pl.multiple_of(base + g, STAGE)
            pltpu.sync_copy(stg.at[pl.ds(0, STAGE)],
                            bucket.at[pl.ds(boff, STAGE)])
            tail = stg[pl.ds(STAGE, L)]
            stg[pl.ds(0, L)] = tail
            return x

        lax.cond(fi >= 1, flush, lambda x: x, 0)
        return qp2 - fi * STAGE, qv2 - fiv * STAGE, g + fi * STAGE, gv + fiv * STAGE

    def step(i, carry, srcbuf, dstbuf):
        qp, qv, g, gv = carry
        d = dstbuf[pl.ds(i * L, L)]
        s = srcbuf[pl.ds(i * L, L)]
        ld = d - lo
        m = (ld >= 0) & (ld < RPW)
        mi = jnp.where(m, 1, 0)
        entries = lax.shift_left(ld, 14) | s
        return compact(mi, entries, stage0, qp, qv, g, gv, qbase)

    def mkstep(sbuf, dbuf2):
        def stepb(i, carry):
            return step(i, carry, sbuf, dbuf2)
        return stepb

    def prefetch(blk, sbuf, dbuf2, ss, sd):
        eoff = pl.multiple_of(blk * SBLK, SBLK)
        pltpu.async_copy(src_hbm.at[pl.ds(eoff, SBLK)], sbuf, ss)
        pltpu.async_copy(dst_hbm.at[pl.ds(eoff, SBLK)], dbuf2, sd)

    def wait(sbuf, dbuf2, ss, sd):
        pltpu.make_async_copy(src_hbm.at[pl.ds(0, SBLK)], sbuf, ss).wait()
        pltpu.make_async_copy(dst_hbm.at[pl.ds(0, SBLK)], dbuf2, sd).wait()

    prefetch(0, srcA, dstA, sas, sad)
    prefetch(1, srcB, dstB, sbs, sbd)

    def blk_body(p, carry):
        wait(srcA, dstA, sas, sad)
        carry = lax.fori_loop(0, SBLK // L, mkstep(srcA, dstA), carry)
        prefetch(2 * p + 2, srcA, dstA, sas, sad)
        wait(srcB, dstB, sbs, sbd)
        carry = lax.fori_loop(0, SBLK // L, mkstep(srcB, dstB), carry)
        prefetch(2 * p + 3, srcB, dstB, sbs, sbd)
        return carry

    carry = lax.fori_loop(0, (NSB - 1) // 2, blk_body, (0, zero, 0, zero))
    wait(srcA, dstA, sas, sad)
    carry = lax.fori_loop(0, SBLK // L, mkstep(srcA, dstA), carry)
    wait(srcB, dstB, sbs, sbd)
    qpA, qvA, gA, gvA = carry

    for t in range(2 * GB // L):
        stage0[pl.ds(qpA + t * L, L)] = sentav
    ndrainA = lax.div(qpA + 2 * GB - 1, 2 * GB) * 2

    @pl.loop(0, ndrainA)
    def _drainA(b):
        soff = pl.multiple_of(b * GB, GB)
        boff = pl.multiple_of(qbase + gA + b * GB, GB)
        pltpu.sync_copy(stage0.at[pl.ds(soff, GB)], bucket.at[pl.ds(boff, GB)])

    nqb = lax.div(gA, GB) + ndrainA

    coff = CR << 14

    def qstep(i, carry, k):
        qp, qv, g, gv = carry
        e = qblkbuf[pl.ds(i * L, L)]
        ld = lax.shift_right_logical(e, 14)
        lk = ld - k * CR
        m = (lk >= 0) & (lk < CR)
        mi = jnp.where(m, 1, 0)
        stg = stage0 if k == 0 else stage1
        return compact(mi, e - k * coff, stg, qp, qv, g, gv,
                       (wid * 3 + k) * ECAP)

    def qblk(b, carry):
        c0, c1 = carry
        boff = pl.multiple_of(qbase + b * GB, GB)
        pltpu.sync_copy(bucket.at[pl.ds(boff, GB)], qblkbuf)
        c0 = lax.fori_loop(0, GB // L, lambda i, c: qstep(i, c, 0), c0)
        c1 = lax.fori_loop(0, GB // L, lambda i, c: qstep(i, c, 1), c1)
        return c0, c1

    z4 = (0, zero, 0, zero)
    (qp0, qv0, g0, gv0), (qp1, qv1, g1, gv1) = lax.fori_loop(
        0, nqb, qblk, (z4, z4))

    for k, qp, g, stg in ((0, qp0, g0, stage0), (1, qp1, g1, stage1)):
        for t in range(2 * GB // L):
            stg[pl.ds(qp + t * L, L)] = sentv
        ndrain = lax.div(qp + 2 * GB - 1, 2 * GB) * 2

        @pl.loop(0, ndrain)
        def _drain(b, _k=k, _g=g, _stg=stg):
            soff = pl.multiple_of(b * GB, GB)
            boff = pl.multiple_of((wid * 3 + _k) * ECAP + _g + b * GB, GB)
            pltpu.sync_copy(_stg.at[pl.ds(soff, GB)],
                            bucket.at[pl.ds(boff, GB)])

    return qv0 + gv0, qv1 + gv1, qp0, g0, qp1, g1


def _sc_scan_accum_body(b_hbm, src_hbm, dst_hbm,
                        sum_o, sq_o, mx_o, mn_o, deg_o, bucket, counts,
                        srcA, dstA, srcB, dstB, sas, sad, sbs, sbd,
                        stage0, stage1, tmp, cbuf,
                        sums, sqs, mxs, mns, degacc,
                        ebufA, sidxA, dbufA, rowA, dsemA,
                        ebufB, sidxB, dbufB, rowB, dsemB):
    wid = lax.axis_index("s") * NC + lax.axis_index("c")
    lane = lax.iota(jnp.int32, L)
    c0v, c1v, qp0, g0, qp1, g1 = _scan_compact(
        wid, src_hbm, dst_hbm, bucket, srcA, dstA, srcB, dstB,
        sas, sad, sbs, sbd, stage0, stage1, tmp, ebufA)
    cbuf[...] = jnp.where(lane == 0, c0v, jnp.where(lane == 1, c1v, 0))
    pltpu.sync_copy(cbuf, counts.at[pl.ds(pl.multiple_of(wid * L, L), L)])
    for k, qp, g in ((0, qp0, g0), (1, qp1, g1)):
        nbat = lax.div(g, GB) + lax.div(qp + 2 * GB - 1, 2 * GB) * 2
        _accum_chunk(wid, k, nbat, b_hbm, bucket,
                     sum_o, sq_o, mx_o, mn_o, deg_o,
                     sums, sqs, mxs, mns, degacc,
                     ebufA, sidxA, dbufA, rowA, dsemA,
                     ebufB, sidxB, dbufB, rowB, dsemB)


def _sc_accum_body(b_hbm, bucket, counts,
                   sum_o, sq_o, mx_o, mn_o, deg_o,
                   cbuf, sums, sqs, mxs, mns, degacc,
                   ebufA, sidxA, dbufA, rowA, dsemA,
                   ebufB, sidxB, dbufB, rowB, dsemB):
    wid = lax.axis_index("s") * NC + lax.axis_index("c")
    pltpu.sync_copy(counts.at[pl.ds(pl.multiple_of(wid * L, L), L)], cbuf)
    cv = cbuf[...]
    for k in (0, 1):
        cnt = cv[k]
        g = lax.div(cnt, STAGE) * STAGE
        qp = cnt - g
        nbat = lax.div(g, GB) + lax.div(qp + 2 * GB - 1, 2 * GB) * 2
        _accum_chunk(wid, k, nbat, b_hbm, bucket,
                     sum_o, sq_o, mx_o, mn_o, deg_o,
                     sums, sqs, mxs, mns, degacc,
                     ebufA, sidxA, dbufA, rowA, dsemA,
                     ebufB, sidxB, dbufB, rowB, dsemB)


def _sc_out_types():
    return [
        jax.ShapeDtypeStruct((NPAD * F,), jnp.float32),
        jax.ShapeDtypeStruct((NPAD * F,), jnp.float32),
        jax.ShapeDtypeStruct((NPAD * F,), jnp.float32),
        jax.ShapeDtypeStruct((NPAD * F,), jnp.float32),
        jax.ShapeDtypeStruct((NW * 2 * DEGR, L), jnp.float32),
    ]


def _sc_accum_scratch():
    return [
        pltpu.VMEM((ACCR * F,), jnp.float32),
        pltpu.VMEM((ACCR * F,), jnp.float32),
        pltpu.VMEM((ACCR * F,), jnp.float32),
        pltpu.VMEM((ACCR * F,), jnp.float32),
        pltpu.VMEM((DEGR, L), jnp.float32),
        pltpu.VMEM((GB,), jnp.int32),
        pltpu.VMEM((GB,), jnp.int32),
        pltpu.VMEM((GB,), jnp.int32),
        pltpu.VMEM((GB, F), jnp.float32),
        pltpu.SemaphoreType.DMA,
        pltpu.VMEM((GB,), jnp.int32),
        pltpu.VMEM((GB,), jnp.int32),
        pltpu.VMEM((GB,), jnp.int32),
        pltpu.VMEM((GB, F), jnp.float32),
        pltpu.SemaphoreType.DMA,
    ]


_SC_MESH = plsc.VectorSubcoreMesh(core_axis_name="c", subcore_axis_name="s")

_sc_scan_accum = functools.partial(
    pl.kernel,
    mesh=_SC_MESH,
    out_type=_sc_out_types() + [
        jax.ShapeDtypeStruct((NW * 3 * ECAP,), jnp.int32),
        jax.ShapeDtypeStruct((NW * L,), jnp.int32),
    ],
    scratch_types=[
        pltpu.VMEM((SBLK,), jnp.int32),
        pltpu.VMEM((SBLK,), jnp.int32),
        pltpu.VMEM((SBLK,), jnp.int32),
        pltpu.VMEM((SBLK,), jnp.int32),
        pltpu.SemaphoreType.DMA,
        pltpu.SemaphoreType.DMA,
        pltpu.SemaphoreType.DMA,
        pltpu.SemaphoreType.DMA,
        pltpu.VMEM((STCAP,), jnp.int32),
        pltpu.VMEM((STCAP,), jnp.int32),
        pltpu.VMEM((2 * L,), jnp.int32),
        pltpu.VMEM((L,), jnp.int32),
    ] + _sc_accum_scratch(),
)(_sc_scan_accum_body)

_sc_accum = functools.partial(
    pl.kernel,
    mesh=_SC_MESH,
    out_type=_sc_out_types(),
    scratch_types=[pltpu.VMEM((L,), jnp.int32)] + _sc_accum_scratch(),
)(_sc_accum_body)


BLK = 512


def _tc_pre_body(x_ref, wcat_ref, bpre_ref, c_ref, b_ref):
    x = x_ref[...]
    w = wcat_ref[...]
    c_ref[...] = jnp.dot(x, w[:, :F], preferred_element_type=jnp.float32) + bpre_ref[...]
    b_ref[...] = jnp.dot(x, w[:, F:], preferred_element_type=jnp.float32)


def _tc_pre(x, wcat, bpre):
    grid = NPAD // BLK
    return pl.pallas_call(
        _tc_pre_body,
        grid=(grid,),
        in_specs=[
            pl.BlockSpec((BLK, F), lambda i: (i, 0)),
            pl.BlockSpec((F, 2 * F), lambda i: (0, 0)),
            pl.BlockSpec((F,), lambda i: (0,)),
        ],
        out_specs=[
            pl.BlockSpec((BLK, F), lambda i: (i, 0)),
            pl.BlockSpec((BLK, F), lambda i: (i, 0)),
        ],
        out_shape=[
            jax.ShapeDtypeStruct((NPAD, F), jnp.float32),
            jax.ShapeDtypeStruct((NPAD, F), jnp.float32),
        ],
    )(x, wcat, bpre)


def _pna_epilogue(x, c, s, sq, mx, mn, deg, wpost, bpost, wlin, blin):
    has = deg > 0.0
    deg_c = jnp.maximum(deg, 1.0)
    sd = s / deg_c
    qd = sq / deg_c
    mean = jnp.where(has, c + sd, 0.0)
    var = qd - sd * sd
    std = jnp.sqrt(jnp.maximum(var, 0.0) + 1e-5)
    mxv = jnp.where(has, c + mx, 0.0)
    mnv = jnp.where(has, c + mn, 0.0)
    agg = jnp.concatenate([mean, mxv, mnv, std], axis=-1)
    lg = jnp.log(deg_c + 1.0)
    amp = lg / AVG_LOG
    att = AVG_LOG / lg
    out = jnp.dot(x, wpost[:F], preferred_element_type=jnp.float32)
    out = out + jnp.dot(agg, wpost[F:5 * F], preferred_element_type=jnp.float32)
    out = out + amp * jnp.dot(agg, wpost[5 * F:9 * F], preferred_element_type=jnp.float32)
    out = out + att * jnp.dot(agg, wpost[9 * F:], preferred_element_type=jnp.float32)
    out = out + bpost
    return jnp.dot(out, wlin, preferred_element_type=jnp.float32) + blin


def _layer_norm(x, g, b, eps=1e-5):
    mu = jnp.mean(x, axis=-1, keepdims=True)
    var = jnp.mean((x - mu) ** 2, axis=-1, keepdims=True)
    return (x - mu) * lax.rsqrt(var + eps) * g + b


def _tc_mid_body(x_ref, c_ref, sum_ref, sq_ref, mx_ref, mn_ref, deg_ref,
                 wpost_ref, bpost_ref, wlin_ref, blin_ref, g_ref, b_ref,
                 wcat2_ref, bpre2_ref,
                 h_ref, c2_ref, b2_ref):
    x = x_ref[...]
    deg = deg_ref[...].reshape(-1, 1)
    out = _pna_epilogue(x, c_ref[...], sum_ref[...], sq_ref[...], mx_ref[...],
                        mn_ref[...], deg, wpost_ref[...], bpost_ref[...],
                        wlin_ref[...], blin_ref[...])
    h = jnp.maximum(_layer_norm(out, g_ref[...], b_ref[...]), 0.0) + x
    h_ref[...] = h
    w2 = wcat2_ref[...]
    c2_ref[...] = jnp.dot(h, w2[:, :F], preferred_element_type=jnp.float32) + bpre2_ref[...]
    b2_ref[...] = jnp.dot(h, w2[:, F:], preferred_element_type=jnp.float32)


def _tc_mid(x, c, s, sq, mx, mn, deg, wpost, bpost, wlin, blin, g, b, wcat2, bpre2):
    grid = NPAD // BLK
    row = lambda: pl.BlockSpec((BLK, F), lambda i: (i, 0))
    full2 = lambda a, bdim: pl.BlockSpec((a, bdim), lambda i: (0, 0))
    vec = pl.BlockSpec((F,), lambda i: (0,))
    return pl.pallas_call(
        _tc_mid_body,
        grid=(grid,),
        in_specs=[
            row(), row(), row(), row(), row(), row(),
            pl.BlockSpec((BLK,), lambda i: (i,)),
            full2(13 * F, F), vec, full2(F, F), vec, vec, vec,
            full2(F, 2 * F), vec,
        ],
        out_specs=[row(), row(), row()],
        out_shape=[
            jax.ShapeDtypeStruct((NPAD, F), jnp.float32),
            jax.ShapeDtypeStruct((NPAD, F), jnp.float32),
            jax.ShapeDtypeStruct((NPAD, F), jnp.float32),
        ],
    )(x, c, s, sq, mx, mn, deg, wpost, bpost, wlin, blin, g, b, wcat2, bpre2)


def _tc_final_body(h_ref, c_ref, sum_ref, sq_ref, mx_ref, mn_ref, deg_ref,
                   wpost_ref, bpost_ref, wlin_ref, blin_ref, g_ref, b_ref,
                   o_ref):
    h = h_ref[...]
    deg = deg_ref[...].reshape(-1, 1)
    out = _pna_epilogue(h, c_ref[...], sum_ref[...], sq_ref[...], mx_ref[...],
                        mn_ref[...], deg, wpost_ref[...], bpost_ref[...],
                        wlin_ref[...], blin_ref[...])
    o_ref[...] = _layer_norm(out, g_ref[...], b_ref[...]) + h


def _tc_final(h, c, s, sq, mx, mn, deg, wpost, bpost, wlin, blin, g, b):
    grid = NPAD // BLK
    row = lambda: pl.BlockSpec((BLK, F), lambda i: (i, 0))
    full2 = lambda a, bdim: pl.BlockSpec((a, bdim), lambda i: (0, 0))
    vec = pl.BlockSpec((F,), lambda i: (0,))
    return pl.pallas_call(
        _tc_final_body,
        grid=(grid,),
        in_specs=[
            row(), row(), row(), row(), row(), row(),
            pl.BlockSpec((BLK,), lambda i: (i,)),
            full2(13 * F, F), vec, full2(F, F), vec, vec, vec,
        ],
        out_specs=row(),
        out_shape=jax.ShapeDtypeStruct((NPAD, F), jnp.float32),
    )(h, c, s, sq, mx, mn, deg, wpost, bpost, wlin, blin, g, b)


def kernel(x, edge_index, Wpre1, bpre1, Wpost1, bpost1, Wlin1, blin1, g1, b1,
           Wpre2, bpre2, Wpost2, bpost2, Wlin2, blin2, g2, b2):
    src = jnp.pad(edge_index[0], (0, 2 * SBLK))
    dst = jnp.pad(edge_index[1], (0, 2 * SBLK))
    x_p = jnp.pad(x, ((0, NPAD - N), (0, 0)))
    wcat1 = jnp.concatenate([Wpre1[:F], Wpre1[F:]], axis=1)
    wcat2 = jnp.concatenate([Wpre2[:F], Wpre2[F:]], axis=1)

    c1, b1t = _tc_pre(x_p, wcat1, bpre1)
    sum1, sq1, mx1, mn1, degr, bucket, counts = _sc_scan_accum(b1t, src, dst)
    sum1, sq1, mx1, mn1 = (a.reshape(NPAD, F) for a in (sum1, sq1, mx1, mn1))
    deg = degr.reshape(NW * 2, DEGR * L)[:, :CR].reshape(NPAD)
    h, c2, b2t = _tc_mid(x_p, c1, sum1, sq1, mx1, mn1, deg,
                         Wpost1, bpost1, Wlin1, blin1, g1, b1, wcat2, bpre2)
    sum2, sq2, mx2, mn2, degr2 = _sc_accum(b2t, bucket, counts)
    sum2, sq2, mx2, mn2 = (a.reshape(NPAD, F) for a in (sum2, sq2, mx2, mn2))
    deg2 = degr2.reshape(NW * 2, DEGR * L)[:, :CR].reshape(NPAD)
    out = _tc_final(h, c2, sum2, sq2, mx2, mn2, deg2,
                    Wpost2, bpost2, Wlin2, blin2, g2, b2)
    return out[:N]


# speedup vs baseline: 1.1897x; 1.1897x over previous
import functools

import jax
import jax.numpy as jnp
import numpy as np
from jax import lax
from jax.experimental import pallas as pl
from jax.experimental.pallas import tpu as pltpu
from jax.experimental.pallas import tpu_sc as plsc

N = 10000
E = 320000
F = 128

AVG_LOG = float(np.log(33.0))

NC = 2
NS = 16
NW = NC * NS
L = 16

NPAD = 10240
RPW = NPAD // NW
CR = RPW // 2
ACCR = CR + 8
DEGR = 16
SBLK = 2560
NSB = E // SBLK
STAGE = 2048
GB = 128
STCAP = STAGE + GB
ECAP = 158 * STAGE
SENT = CR << 14
SENTA = RPW << 14


def _accum_chunk(wid, k, nbat, b_hbm, bucket, sum_o, sq_o, mx_o, mn_o, deg_o,
                 sums, sqs, mxs, mns, degacc,
                 ebufA, sidxA, dbufA, rowA, dsemA):
    zero = jnp.zeros((L,), jnp.float32)
    neg = jnp.full((L,), -3.0e38, jnp.float32)
    pos = jnp.full((L,), 3.0e38, jnp.float32)
    lane = lax.iota(jnp.int32, L)
    bbase = (wid * 3 + k) * ECAP

    @pl.loop(0, DEGR)
    def _initd(r):
        degacc[r, :] = zero

    @pl.loop(0, ACCR)
    def _init(r):
        rb = r * F
        for j in range(F // L):
            sl = pl.ds(rb + j * L, L)
            sums[sl] = zero
            sqs[sl] = zero
            mxs[sl] = neg
            mns[sl] = pos

    @pl.loop(0, nbat)
    def _batch(b):
        off = pl.multiple_of(bbase + b * GB, GB)
        pltpu.sync_copy(bucket.at[pl.ds(off, GB)], ebufA)
        for i in range(GB // L):
            v = ebufA[pl.ds(i * L, L)]
            sidxA[pl.ds(i * L, L)] = v & 16383
            dbufA[pl.ds(i * L, L)] = lax.shift_right_logical(v, 14)
        pltpu.async_copy(b_hbm.at[sidxA], rowA, dsemA).wait()

        @pl.loop(0, GB // L)
        def _grp(gi):
            dv = dbufA[pl.ds(gi * L, L)]
            for l in range(L):
                dle = dv[l]
                e = gi * L + l
                db = dle * F
                dg = lax.shift_right_logical(dle, 4)
                onehot = jnp.where(lane == (dle & 15), 1.0, 0.0)
                degacc[dg, :] = degacc[dg, :] + onehot
                for j in range(F // L):
                    sl = pl.ds(db + j * L, L)
                    r = rowA[e, pl.ds(j * L, L)]
                    sums[sl] = sums[sl] + r
                    sqs[sl] = sqs[sl] + r * r
                    mxs[sl] = jnp.maximum(mxs[sl], r)
                    mns[sl] = jnp.minimum(mns[sl], r)

    base = pl.multiple_of((wid * RPW + k * CR) * F, CR * F)
    pltpu.sync_copy(sums.at[pl.ds(0, CR * F)], sum_o.at[pl.ds(base, CR * F)])
    pltpu.sync_copy(sqs.at[pl.ds(0, CR * F)], sq_o.at[pl.ds(base, CR * F)])
    pltpu.sync_copy(mxs.at[pl.ds(0, CR * F)], mx_o.at[pl.ds(base, CR * F)])
    pltpu.sync_copy(mns.at[pl.ds(0, CR * F)], mn_o.at[pl.ds(base, CR * F)])
    dbase = pl.multiple_of((wid * 2 + k) * DEGR, DEGR)
    pltpu.sync_copy(degacc.at[pl.ds(0, DEGR)], deg_o.at[pl.ds(dbase, DEGR)])


def _scan_compact(wid, src_hbm, dst_hbm, bucket, srcA, dstA, srcB, dstB,
                  sas, sad, sbs, sbd, stage0, stage1, tmp, qblkbuf):
    lo = wid * RPW
    lane = lax.iota(jnp.int32, L)
    zero = jnp.zeros((L,), jnp.int32)
    stage_v = jnp.full((L,), STAGE, jnp.int32)
    sentv = jnp.full((L,), SENT, jnp.int32)
    sentav = jnp.full((L,), SENTA, jnp.int32)
    tmp[pl.ds(0, L)] = zero
    qbase = (wid * 3 + 2) * ECAP

    def compact(mi, values, stg, qp, qv, g, gv, base):
        acc = mi
        for sh in (1, 2, 4, 8):
            tmp[pl.ds(L, L)] = acc
            acc = acc + tmp[pl.ds(L - sh, L)]
        target = lane + 1
        posv = zero
        for stp in (8, 4, 2, 1):
            cand = posv + stp
            pv = acc[(cand - 1) & 15]
            posv = jnp.where((pv < target) & (cand <= L), cand, posv)
        stg[pl.ds(qp, L)] = values[posv & 15]
        c15 = acc[15]
        qp2 = qp + c15
        qv2 = qv + c15
        fi = lax.div(qp2, STAGE)
        fiv = lax.div(qv2, stage_v)

        def flush(x):
            boff = pl.multiple_of(base + g, STAGE)
            pltpu.sync_copy(stg.at[pl.ds(0, STAGE)],
                            bucket.at[pl.ds(boff, STAGE)])
            tail = stg[pl.ds(STAGE, L)]
            stg[pl.ds(0, L)] = tail
            return x

        lax.cond(fi >= 1, flush, lambda x: x, 0)
        return qp2 - fi * STAGE, qv2 - fiv * STAGE, g + fi * STAGE, gv + fiv * STAGE

    def step(i, carry, srcbuf, dstbuf):
        qp, qv, g, gv = carry
        d = dstbuf[pl.ds(i * L, L)]
        s = srcbuf[pl.ds(i * L, L)]
        ld = d - lo
        m = (ld >= 0) & (ld < RPW)
        mi = jnp.where(m, 1, 0)
        entries = lax.shift_left(ld, 14) | s
        return compact(mi, entries, stage0, qp, qv, g, gv, qbase)

    def mkstep(sbuf, dbuf2):
        def stepb(i, carry):
            return step(i, carry, sbuf, dbuf2)
        return stepb

    def prefetch(blk, sbuf, dbuf2, ss, sd):
        eoff = pl.multiple_of(blk * SBLK, SBLK)
        pltpu.async_copy(src_hbm.at[pl.ds(eoff, SBLK)], sbuf, ss)
        pltpu.async_copy(dst_hbm.at[pl.ds(eoff, SBLK)], dbuf2, sd)

    def wait(sbuf, dbuf2, ss, sd):
        pltpu.make_async_copy(src_hbm.at[pl.ds(0, SBLK)], sbuf, ss).wait()
        pltpu.make_async_copy(dst_hbm.at[pl.ds(0, SBLK)], dbuf2, sd).wait()

    prefetch(0, srcA, dstA, sas, sad)
    prefetch(1, srcB, dstB, sbs, sbd)

    def blk_body(p, carry):
        wait(srcA, dstA, sas, sad)
        carry = lax.fori_loop(0, SBLK // L, mkstep(srcA, dstA), carry)
        prefetch(2 * p + 2, srcA, dstA, sas, sad)
        wait(srcB, dstB, sbs, sbd)
        carry = lax.fori_loop(0, SBLK // L, mkstep(srcB, dstB), carry)
        prefetch(2 * p + 3, srcB, dstB, sbs, sbd)
        return carry

    carry = lax.fori_loop(0, (NSB - 1) // 2, blk_body, (0, zero, 0, zero))
    wait(srcA, dstA, sas, sad)
    carry = lax.fori_loop(0, SBLK // L, mkstep(srcA, dstA), carry)
    wait(srcB, dstB, sbs, sbd)
    qpA, qvA, gA, gvA = carry

    for t in range(GB // L):
        stage0[pl.ds(qpA + t * L, L)] = sentav
    ndrainA = lax.div(qpA + GB - 1, GB)

    @pl.loop(0, ndrainA)
    def _drainA(b):
        soff = pl.multiple_of(b * GB, GB)
        boff = pl.multiple_of(qbase + gA + b * GB, GB)
        pltpu.sync_copy(stage0.at[pl.ds(soff, GB)], bucket.at[pl.ds(boff, GB)])

    nqb = lax.div(gA, GB) + ndrainA

    coff = CR << 14

    def qstep(i, carry, k):
        qp, qv, g, gv = carry
        e = qblkbuf[pl.ds(i * L, L)]
        ld = lax.shift_right_logical(e, 14)
        lk = ld - k * CR
        m = (lk >= 0) & (lk < CR)
        mi = jnp.where(m, 1, 0)
        stg = stage0 if k == 0 else stage1
        return compact(mi, e - k * coff, stg, qp, qv, g, gv,
                       (wid * 3 + k) * ECAP)

    def qblk(b, carry):
        c0, c1 = carry
        boff = pl.multiple_of(qbase + b * GB, GB)
        pltpu.sync_copy(bucket.at[pl.ds(boff, GB)], qblkbuf)
        c0 = lax.fori_loop(0, GB // L, lambda i, c: qstep(i, c, 0), c0)
        c1 = lax.fori_loop(0, GB // L, lambda i, c: qstep(i, c, 1), c1)
        return c0, c1

    z4 = (0, zero, 0, zero)
    (qp0, qv0, g0, gv0), (qp1, qv1, g1, gv1) = lax.fori_loop(
        0, nqb, qblk, (z4, z4))

    for k, qp, g, stg in ((0, qp0, g0, stage0), (1, qp1, g1, stage1)):
        for t in range(GB // L):
            stg[pl.ds(qp + t * L, L)] = sentv
        ndrain = lax.div(qp + GB - 1, GB)

        @pl.loop(0, ndrain)
        def _drain(b, _k=k, _g=g, _stg=stg):
            soff = pl.multiple_of(b * GB, GB)
            boff = pl.multiple_of((wid * 3 + _k) * ECAP + _g + b * GB, GB)
            pltpu.sync_copy(_stg.at[pl.ds(soff, GB)],
                            bucket.at[pl.ds(boff, GB)])

    return qv0 + gv0, qv1 + gv1, qp0, g0, qp1, g1


def _sc_scan_accum_body(b_hbm, src_hbm, dst_hbm,
                        sum_o, sq_o, mx_o, mn_o, deg_o, bucket, counts,
                        srcA, dstA, srcB, dstB, sas, sad, sbs, sbd,
                        stage0, stage1, tmp, cbuf,
                        sums, sqs, mxs, mns, degacc,
                        ebufA, sidxA, dbufA, rowA, dsemA):
    wid = lax.axis_index("s") * NC + lax.axis_index("c")
    lane = lax.iota(jnp.int32, L)
    c0v, c1v, qp0, g0, qp1, g1 = _scan_compact(
        wid, src_hbm, dst_hbm, bucket, srcA, dstA, srcB, dstB,
        sas, sad, sbs, sbd, stage0, stage1, tmp, ebufA)
    cbuf[...] = jnp.where(lane == 0, c0v, jnp.where(lane == 1, c1v, 0))
    pltpu.sync_copy(cbuf, counts.at[pl.ds(pl.multiple_of(wid * L, L), L)])
    for k, qp, g in ((0, qp0, g0), (1, qp1, g1)):
        nbat = lax.div(g, GB) + lax.div(qp + GB - 1, GB)
        _accum_chunk(wid, k, nbat, b_hbm, bucket,
                     sum_o, sq_o, mx_o, mn_o, deg_o,
                     sums, sqs, mxs, mns, degacc,
                     ebufA, sidxA, dbufA, rowA, dsemA)


def _sc_accum_body(b_hbm, bucket, counts,
                   sum_o, sq_o, mx_o, mn_o, deg_o,
                   cbuf, sums, sqs, mxs, mns, degacc,
                   ebufA, sidxA, dbufA, rowA, dsemA):
    wid = lax.axis_index("s") * NC + lax.axis_index("c")
    pltpu.sync_copy(counts.at[pl.ds(pl.multiple_of(wid * L, L), L)], cbuf)
    cv = cbuf[...]
    for k in (0, 1):
        cnt = cv[k]
        g = lax.div(cnt, STAGE) * STAGE
        qp = cnt - g
        nbat = lax.div(g, GB) + lax.div(qp + GB - 1, GB)
        _accum_chunk(wid, k, nbat, b_hbm, bucket,
                     sum_o, sq_o, mx_o, mn_o, deg_o,
                     sums, sqs, mxs, mns, degacc,
                     ebufA, sidxA, dbufA, rowA, dsemA)


def _sc_out_types():
    return [
        jax.ShapeDtypeStruct((NPAD * F,), jnp.float32),
        jax.ShapeDtypeStruct((NPAD * F,), jnp.float32),
        jax.ShapeDtypeStruct((NPAD * F,), jnp.float32),
        jax.ShapeDtypeStruct((NPAD * F,), jnp.float32),
        jax.ShapeDtypeStruct((NW * 2 * DEGR, L), jnp.float32),
    ]


def _sc_accum_scratch():
    return [
        pltpu.VMEM((ACCR * F,), jnp.float32),
        pltpu.VMEM((ACCR * F,), jnp.float32),
        pltpu.VMEM((ACCR * F,), jnp.float32),
        pltpu.VMEM((ACCR * F,), jnp.float32),
        pltpu.VMEM((DEGR, L), jnp.float32),
        pltpu.VMEM((GB,), jnp.int32),
        pltpu.VMEM((GB,), jnp.int32),
        pltpu.VMEM((GB,), jnp.int32),
        pltpu.VMEM((GB, F), jnp.float32),
        pltpu.SemaphoreType.DMA,
    ]


_SC_MESH = plsc.VectorSubcoreMesh(core_axis_name="c", subcore_axis_name="s")

_sc_scan_accum = functools.partial(
    pl.kernel,
    mesh=_SC_MESH,
    out_type=_sc_out_types() + [
        jax.ShapeDtypeStruct((NW * 3 * ECAP,), jnp.int32),
        jax.ShapeDtypeStruct((NW * L,), jnp.int32),
    ],
    scratch_types=[
        pltpu.VMEM((SBLK,), jnp.int32),
        pltpu.VMEM((SBLK,), jnp.int32),
        pltpu.VMEM((SBLK,), jnp.int32),
        pltpu.VMEM((SBLK,), jnp.int32),
        pltpu.SemaphoreType.DMA,
        pltpu.SemaphoreType.DMA,
        pltpu.SemaphoreType.DMA,
        pltpu.SemaphoreType.DMA,
        pltpu.VMEM((STCAP,), jnp.int32),
        pltpu.VMEM((STCAP,), jnp.int32),
        pltpu.VMEM((2 * L,), jnp.int32),
        pltpu.VMEM((L,), jnp.int32),
    ] + _sc_accum_scratch(),
)(_sc_scan_accum_body)

_sc_accum = functools.partial(
    pl.kernel,
    mesh=_SC_MESH,
    out_type=_sc_out_types(),
    scratch_types=[pltpu.VMEM((L,), jnp.int32)] + _sc_accum_scratch(),
)(_sc_accum_body)


BLK = 512


def _tc_pre_body(x_ref, wcat_ref, bpre_ref, c_ref, b_ref):
    x = x_ref[...]
    w = wcat_ref[...]
    c_ref[...] = jnp.dot(x, w[:, :F], preferred_element_type=jnp.float32) + bpre_ref[...]
    b_ref[...] = jnp.dot(x, w[:, F:], preferred_element_type=jnp.float32)


def _tc_pre(x, wcat, bpre):
    grid = NPAD // BLK
    return pl.pallas_call(
        _tc_pre_body,
        grid=(grid,),
        in_specs=[
            pl.BlockSpec((BLK, F), lambda i: (i, 0)),
            pl.BlockSpec((F, 2 * F), lambda i: (0, 0)),
            pl.BlockSpec((F,), lambda i: (0,)),
        ],
        out_specs=[
            pl.BlockSpec((BLK, F), lambda i: (i, 0)),
            pl.BlockSpec((BLK, F), lambda i: (i, 0)),
        ],
        out_shape=[
            jax.ShapeDtypeStruct((NPAD, F), jnp.float32),
            jax.ShapeDtypeStruct((NPAD, F), jnp.float32),
        ],
    )(x, wcat, bpre)


def _pna_epilogue(x, c, s, sq, mx, mn, deg, wpost, bpost, wlin, blin):
    has = deg > 0.0
    deg_c = jnp.maximum(deg, 1.0)
    sd = s / deg_c
    qd = sq / deg_c
    mean = jnp.where(has, c + sd, 0.0)
    var = qd - sd * sd
    std = jnp.sqrt(jnp.maximum(var, 0.0) + 1e-5)
    mxv = jnp.where(has, c + mx, 0.0)
    mnv = jnp.where(has, c + mn, 0.0)
    agg = jnp.concatenate([mean, mxv, mnv, std], axis=-1)
    lg = jnp.log(deg_c + 1.0)
    amp = lg / AVG_LOG
    att = AVG_LOG / lg
    out = jnp.dot(x, wpost[:F], preferred_element_type=jnp.float32)
    out = out + jnp.dot(agg, wpost[F:5 * F], preferred_element_type=jnp.float32)
    out = out + amp * jnp.dot(agg, wpost[5 * F:9 * F], preferred_element_type=jnp.float32)
    out = out + att * jnp.dot(agg, wpost[9 * F:], preferred_element_type=jnp.float32)
    out = out + bpost
    return jnp.dot(out, wlin, preferred_element_type=jnp.float32) + blin


def _layer_norm(x, g, b, eps=1e-5):
    mu = jnp.mean(x, axis=-1, keepdims=True)
    var = jnp.mean((x - mu) ** 2, axis=-1, keepdims=True)
    return (x - mu) * lax.rsqrt(var + eps) * g + b


def _tc_mid_body(x_ref, c_ref, sum_ref, sq_ref, mx_ref, mn_ref, deg_ref,
                 wpost_ref, bpost_ref, wlin_ref, blin_ref, g_ref, b_ref,
                 wcat2_ref, bpre2_ref,
                 h_ref, c2_ref, b2_ref):
    x = x_ref[...]
    deg = deg_ref[...].reshape(-1, 1)
    out = _pna_epilogue(x, c_ref[...], sum_ref[...], sq_ref[...], mx_ref[...],
                        mn_ref[...], deg, wpost_ref[...], bpost_ref[...],
                        wlin_ref[...], blin_ref[...])
    h = jnp.maximum(_layer_norm(out, g_ref[...], b_ref[...]), 0.0) + x
    h_ref[...] = h
    w2 = wcat2_ref[...]
    c2_ref[...] = jnp.dot(h, w2[:, :F], preferred_element_type=jnp.float32) + bpre2_ref[...]
    b2_ref[...] = jnp.dot(h, w2[:, F:], preferred_element_type=jnp.float32)


def _tc_mid(x, c, s, sq, mx, mn, deg, wpost, bpost, wlin, blin, g, b, wcat2, bpre2):
    grid = NPAD // BLK
    row = lambda: pl.BlockSpec((BLK, F), lambda i: (i, 0))
    full2 = lambda a, bdim: pl.BlockSpec((a, bdim), lambda i: (0, 0))
    vec = pl.BlockSpec((F,), lambda i: (0,))
    return pl.pallas_call(
        _tc_mid_body,
        grid=(grid,),
        in_specs=[
            row(), row(), row(), row(), row(), row(),
            pl.BlockSpec((BLK,), lambda i: (i,)),
            full2(13 * F, F), vec, full2(F, F), vec, vec, vec,
            full2(F, 2 * F), vec,
        ],
        out_specs=[row(), row(), row()],
        out_shape=[
            jax.ShapeDtypeStruct((NPAD, F), jnp.float32),
            jax.ShapeDtypeStruct((NPAD, F), jnp.float32),
            jax.ShapeDtypeStruct((NPAD, F), jnp.float32),
        ],
    )(x, c, s, sq, mx, mn, deg, wpost, bpost, wlin, blin, g, b, wcat2, bpre2)


def _tc_final_body(h_ref, c_ref, sum_ref, sq_ref, mx_ref, mn_ref, deg_ref,
                   wpost_ref, bpost_ref, wlin_ref, blin_ref, g_ref, b_ref,
                   o_ref):
    h = h_ref[...]
    deg = deg_ref[...].reshape(-1, 1)
    out = _pna_epilogue(h, c_ref[...], sum_ref[...], sq_ref[...], mx_ref[...],
                        mn_ref[...], deg, wpost_ref[...], bpost_ref[...],
                        wlin_ref[...], blin_ref[...])
    o_ref[...] = _layer_norm(out, g_ref[...], b_ref[...]) + h


def _tc_final(h, c, s, sq, mx, mn, deg, wpost, bpost, wlin, blin, g, b):
    grid = NPAD // BLK
    row = lambda: pl.BlockSpec((BLK, F), lambda i: (i, 0))
    full2 = lambda a, bdim: pl.BlockSpec((a, bdim), lambda i: (0, 0))
    vec = pl.BlockSpec((F,), lambda i: (0,))
    return pl.pallas_call(
        _tc_final_body,
        grid=(grid,),
        in_specs=[
            row(), row(), row(), row(), row(), row(),
            pl.BlockSpec((BLK,), lambda i: (i,)),
            full2(13 * F, F), vec, full2(F, F), vec, vec, vec,
        ],
        out_specs=row(),
        out_shape=jax.ShapeDtypeStruct((NPAD, F), jnp.float32),
    )(h, c, s, sq, mx, mn, deg, wpost, bpost, wlin, blin, g, b)


def kernel(x, edge_index, Wpre1, bpre1, Wpost1, bpost1, Wlin1, blin1, g1, b1,
           Wpre2, bpre2, Wpost2, bpost2, Wlin2, blin2, g2, b2):
    src = jnp.pad(edge_index[0], (0, 2 * SBLK))
    dst = jnp.pad(edge_index[1], (0, 2 * SBLK))
    x_p = jnp.pad(x, ((0, NPAD - N), (0, 0)))
    wcat1 = jnp.concatenate([Wpre1[:F], Wpre1[F:]], axis=1)
    wcat2 = jnp.concatenate([Wpre2[:F], Wpre2[F:]], axis=1)

    c1, b1t = _tc_pre(x_p, wcat1, bpre1)
    sum1, sq1, mx1, mn1, degr, bucket, counts = _sc_scan_accum(b1t, src, dst)
    sum1, sq1, mx1, mn1 = (a.reshape(NPAD, F) for a in (sum1, sq1, mx1, mn1))
    deg = degr.reshape(NW * 2, DEGR * L)[:, :CR].reshape(NPAD)
    h, c2, b2t = _tc_mid(x_p, c1, sum1, sq1, mx1, mn1, deg,
                         Wpost1, bpost1, Wlin1, blin1, g1, b1, wcat2, bpre2)
    sum2, sq2, mx2, mn2, degr2 = _sc_accum(b2t, bucket, counts)
    sum2, sq2, mx2, mn2 = (a.reshape(NPAD, F) for a in (sum2, sq2, mx2, mn2))
    deg2 = degr2.reshape(NW * 2, DEGR * L)[:, :CR].reshape(NPAD)
    out = _tc_final(h, c2, sum2, sq2, mx2, mn2, deg2,
                    Wpost2, bpost2, Wlin2, blin2, g2, b2)
    return out[:N]
